# baseline (speedup 1.0000x reference)
import functools

import jax
import jax.numpy as jnp
from jax import lax
from jax.experimental import pallas as pl
from jax.experimental.pallas import tpu as pltpu
from jax.experimental.pallas import tpu_sc as plsc

NSEL = 300
NPAD = 304
B = 32
N = 256000
CHUNK = 32000
NCHUNK = N // CHUNK
VEC_PER_CHUNK = CHUNK // 16
H0 = 2048
H1 = 2048
H2 = 1024
INT_MIN = -2147483648
CAP = 4096


def _iota16():
    return lax.broadcasted_iota(jnp.int32, (16,), 0)


def _splat_last(v):
    return lax.gather(
        v,
        jnp.full((16, 1), 15, jnp.int32),
        lax.GatherDimensionNumbers(
            offset_dims=(), collapsed_slice_dims=(0,), start_index_map=(0,)),
        (1,),
        mode=lax.GatherScatterMode.PROMISE_IN_BOUNDS,
    )


def _to_key(x_f32):
    s = plsc.bitcast(x_f32, jnp.int32)
    m = lax.shift_right_arithmetic(s, 31)
    return s ^ lax.shift_right_logical(m, 1)


def _from_key(k_i32):
    s = k_i32 ^ lax.shift_right_logical(lax.shift_right_arithmetic(k_i32, 31), 1)
    return plsc.bitcast(s, jnp.float32)


def _zero_hist(hist, nbins):
    del nbins
    def body(r, _):
        hist[pl.ds(r * 16, 16)] = jnp.zeros((16,), jnp.int32)
        return 0

    lax.fori_loop(0, 2049, body, 0)


def _digit_select(hist, nbins, needed):
    lane = _iota16()

    def body(t, carry):
        suffix, b, nab = carry
        d = (nbins - 1) - t
        row = plsc.load_gather(hist, [d + lane * 2049])
        new_suffix = suffix + jnp.sum(row)
        hit = (suffix < needed) & (new_suffix >= needed)
        b = jnp.where(hit, d, b)
        nab = jnp.where(hit, suffix, nab)
        return (new_suffix, b, nab)

    _, b, nab = lax.fori_loop(0, nbins, body, (jnp.int32(0), jnp.int32(0), jnp.int32(0)))
    return b, nab


def _sc_body(lg_hbm, bbox_hbm, mask_hbm, probs_out, caps_out, bbox_out, masks_out,
             buf, hist, selk, seli, candk, candi, sortk, sorti,
             probbuf, capbuf, maskoutbuf, bboxoutbuf, bboxbuf, maskbuf):
    nc = 2
    b = lax.axis_index("s") * nc + lax.axis_index("c")
    lane = _iota16()
    ones = jnp.ones((16,), jnp.int32)

    pltpu.sync_copy(bbox_hbm.at[b], bboxbuf)
    pltpu.sync_copy(mask_hbm.at[b], maskbuf)

    _zero_hist(hist, H0)
    for c in range(NCHUNK):
        pltpu.sync_copy(lg_hbm.at[b * NCHUNK + c], buf)

        def p1(i, _):
            base = i * 128
            for u in range(8):
                key = _to_key(buf[pl.ds(base + u * 16, 16)])
                d0 = lax.shift_right_logical(key ^ INT_MIN, 21)
                plsc.addupdate_scatter(hist, [d0 + lane * 2049], ones)
            return 0

        lax.fori_loop(0, VEC_PER_CHUNK // 8, p1, 0)
    b0, n_above0 = _digit_select(hist, H0, jnp.int32(NSEL))

    g_s = jnp.zeros((16,), jnp.int32)
    nb_s = jnp.zeros((16,), jnp.int32)
    for c in range(NCHUNK):
        pltpu.sync_copy(lg_hbm.at[b * NCHUNK + c], buf)

        def p2(i, carry):
            g_s, nb_s = carry
            base = i * 64
            for u in range(4):
                key = _to_key(buf[pl.ds(base + u * 16, 16)])
                kb = key ^ INT_MIN
                idx = c * CHUNK + base + u * 16 + lane
                d0 = lax.shift_right_logical(kb, 21)
                m_gt = d0 > b0
                run = g_s + plsc.cumsum(m_gt.astype(jnp.int32))
                pos = jnp.clip(run - 1, 0, 319)
                plsc.store_scatter(selk, [pos], key, mask=m_gt)
                plsc.store_scatter(seli, [pos], idx, mask=m_gt)
                g_s = _splat_last(run)
                m_b = d0 == b0
                run_b = nb_s + plsc.cumsum(m_b.astype(jnp.int32))
                pos_b = jnp.clip(run_b - 1, 0, CAP - 1)
                plsc.store_scatter(candk, [pos_b], key, mask=m_b)
                plsc.store_scatter(candi, [pos_b], idx, mask=m_b)
                nb_s = _splat_last(run_b)
            return (g_s, nb_s)

        g_s, nb_s = lax.fori_loop(0, VEC_PER_CHUNK // 4, p2, (g_s, nb_s))

    g = jnp.max(g_s)
    nb = jnp.max(nb_s)

    needed1 = jnp.int32(NSEL) - n_above0

    @pl.when(nb <= CAP)
    def _refine_local():
        _zero_hist(hist, H1)

        def r1(i, _):
            valid = (i * 16 + lane) < nb
            kb = candk[pl.ds(i * 16, 16)] ^ INT_MIN
            d1 = lax.shift_right_logical(kb, 10) & 0x7FF
            plsc.addupdate_scatter(hist, [d1 + lane * 2049], ones, mask=valid)
            return 0

        lax.fori_loop(0, CAP // 16, r1, 0)
        b1, gt1 = _digit_select(hist, H1, needed1)
        n_above1 = n_above0 + gt1

        _zero_hist(hist, H2)

        def r2(i, gg):
            valid = (i * 16 + lane) < nb
            key = candk[pl.ds(i * 16, 16)]
            idx = candi[pl.ds(i * 16, 16)]
            kb = key ^ INT_MIN
            d1 = lax.shift_right_logical(kb, 10) & 0x7FF
            m_gt = valid & (d1 > b1)
            csum = plsc.cumsum(m_gt.astype(jnp.int32))
            pos = jnp.clip(gg + csum - 1, 0, 319)
            plsc.store_scatter(selk, [pos], key, mask=m_gt)
            plsc.store_scatter(seli, [pos], idx, mask=m_gt)
            m_eq = valid & (d1 == b1)
            d2 = kb & 0x3FF
            plsc.addupdate_scatter(hist, [d2 + lane * 2049], ones, mask=m_eq)
            return gg + jnp.sum(m_gt.astype(jnp.int32))

        gg = lax.fori_loop(0, CAP // 16, r2, g)
        b2, gt2 = _digit_select(hist, H2, jnp.int32(NSEL) - n_above1)
        n_gt = n_above1 + gt2
        thr = (lax.shift_left(b0, 21) | lax.shift_left(b1, 10) | b2) ^ INT_MIN

        def r3(i, carry):
            gg, e = carry
            valid = (i * 16 + lane) < nb
            key = candk[pl.ds(i * 16, 16)]
            idx = candi[pl.ds(i * 16, 16)]
            kb = key ^ INT_MIN
            d1 = lax.shift_right_logical(kb, 10) & 0x7FF
            m_gt = valid & (d1 == b1) & (key > thr)
            csum = plsc.cumsum(m_gt.astype(jnp.int32))
            pos = jnp.clip(gg + csum - 1, 0, 319)
            plsc.store_scatter(selk, [pos], key, mask=m_gt)
            plsc.store_scatter(seli, [pos], idx, mask=m_gt)
            gg = gg + jnp.sum(m_gt.astype(jnp.int32))
            m_eq = valid & (key == thr)
            csum_e = plsc.cumsum(m_eq.astype(jnp.int32))
            take = m_eq & ((e + csum_e) <= NSEL)
            pos_e = jnp.clip(e + csum_e - 1, 0, 319)
            plsc.store_scatter(selk, [pos_e], key, mask=take)
            plsc.store_scatter(seli, [pos_e], idx, mask=take)
            return (gg, e + jnp.sum(take.astype(jnp.int32)))

        lax.fori_loop(0, CAP // 16, r3, (n_above1, n_gt))

    @pl.when(nb > CAP)
    def _refine_streamed():
        _zero_hist(hist, H1)
        for c in range(NCHUNK):
            pltpu.sync_copy(lg_hbm.at[b * NCHUNK + c], buf)

            def f1(i, _):
                kb = _to_key(buf[pl.ds(i * 16, 16)]) ^ INT_MIN
                d0 = lax.shift_right_logical(kb, 21)
                d1 = lax.shift_right_logical(kb, 10) & 0x7FF
                plsc.addupdate_scatter(hist, [d1 + lane * 2049], ones,
                                       mask=d0 == b0)
                return 0

            lax.fori_loop(0, VEC_PER_CHUNK, f1, 0)
        b1, gt1 = _digit_select(hist, H1, needed1)
        n_above1 = n_above0 + gt1
        p21 = lax.shift_left(b0, 11) | b1

        _zero_hist(hist, H2)
        gg = g
        for c in range(NCHUNK):
            pltpu.sync_copy(lg_hbm.at[b * NCHUNK + c], buf)

            def f2(i, gg):
                key = _to_key(buf[pl.ds(i * 16, 16)])
                kb = key ^ INT_MIN
                idx = c * CHUNK + i * 16 + lane
                pfx = lax.shift_right_logical(kb, 10)
                m_gt = (pfx > p21) & (lax.shift_right_logical(kb, 21) == b0)
                csum = plsc.cumsum(m_gt.astype(jnp.int32))
                pos = jnp.clip(gg + csum - 1, 0, 319)
                plsc.store_scatter(selk, [pos], key, mask=m_gt)
                plsc.store_scatter(seli, [pos], idx, mask=m_gt)
                m_eq = pfx == p21
                d2 = kb & 0x3FF
                plsc.addupdate_scatter(hist, [d2 + lane * 2049], ones, mask=m_eq)
                return gg + jnp.sum(m_gt.astype(jnp.int32))

            gg = lax.fori_loop(0, VEC_PER_CHUNK, f2, gg)
        b2, gt2 = _digit_select(hist, H2, jnp.int32(NSEL) - n_above1)
        n_gt = n_above1 + gt2
        thr = (lax.shift_left(p21, 10) | b2) ^ INT_MIN

        e = n_gt
        for c in range(NCHUNK):
            pltpu.sync_copy(lg_hbm.at[b * NCHUNK + c], buf)

            def f3(i, carry):
                gg, e = carry
                key = _to_key(buf[pl.ds(i * 16, 16)])
                kb = key ^ INT_MIN
                idx = c * CHUNK + i * 16 + lane
                pfx = lax.shift_right_logical(kb, 10)
                m_gt = (pfx == p21) & (key > thr)
                csum = plsc.cumsum(m_gt.astype(jnp.int32))
                pos = jnp.clip(gg + csum - 1, 0, 319)
                plsc.store_scatter(selk, [pos], key, mask=m_gt)
                plsc.store_scatter(seli, [pos], idx, mask=m_gt)
                gg = gg + jnp.sum(m_gt.astype(jnp.int32))
                m_eq = key == thr
                csum_e = plsc.cumsum(m_eq.astype(jnp.int32))
                take = m_eq & ((e + csum_e) <= NSEL)
                pos_e = jnp.clip(e + csum_e - 1, 0, 319)
                plsc.store_scatter(selk, [pos_e], key, mask=take)
                plsc.store_scatter(seli, [pos_e], idx, mask=take)
                return (gg, e + jnp.sum(take.astype(jnp.int32)))

            gg, e = lax.fori_loop(0, VEC_PER_CHUNK, f3, (gg, e))

    cur_k = selk[pl.ds(288, 16)]
    cur_i = seli[pl.ds(288, 16)]
    m_pad = lane >= 12
    selk[pl.ds(288, 16)] = jnp.where(m_pad, jnp.full((16,), INT_MIN, jnp.int32), cur_k)
    seli[pl.ds(288, 16)] = jnp.where(m_pad, 0x4000000 + lane, cur_i)

    sortk[pl.ds(288, 16)] = jnp.zeros((16,), jnp.int32)
    sorti[pl.ds(288, 16)] = jnp.zeros((16,), jnp.int32)

    for iv in range(NPAD // 16):
        ki = selk[pl.ds(iv * 16, 16)]
        ii = seli[pl.ds(iv * 16, 16)]

        def rank_body(j, rank):
            jb = jnp.full((16,), j, jnp.int32)
            kj = plsc.load_gather(selk, [jb])
            ij = plsc.load_gather(seli, [jb])
            beats = (kj > ki) | ((kj == ki) & (ij < ii))
            return rank + beats.astype(jnp.int32)

        rank = lax.fori_loop(0, NPAD, rank_body, jnp.zeros((16,), jnp.int32))
        m_real = rank < NSEL
        plsc.store_scatter(sortk, [rank], ki, mask=m_real)
        plsc.store_scatter(sorti, [rank], ii, mask=m_real)

    for v in range(NPAD // 16):
        sl = pl.ds(v * 16, 16)
        k = sortk[sl]
        idx = sorti[sl]
        logit = _from_key(k)
        probbuf[sl] = 1.0 / (1.0 + jnp.exp(-logit))
        capbuf[sl] = idx & 255
        box = jnp.minimum(lax.shift_right_logical(idx, 8), 999)
        maskoutbuf[sl] = plsc.load_gather(maskbuf, [box])

    for v in range(1200 // 16):
        bi = v * 4 + lax.shift_right_logical(lane, 2)
        si = plsc.load_gather(sorti, [bi])
        box = jnp.minimum(lax.shift_right_logical(si, 8), 999)
        addr = lax.shift_left(box, 2) | (lane & 3)
        bboxoutbuf[pl.ds(v * 16, 16)] = plsc.load_gather(bboxbuf, [addr])

    pltpu.sync_copy(probbuf, probs_out.at[b])
    pltpu.sync_copy(capbuf, caps_out.at[b])
    pltpu.sync_copy(bboxoutbuf, bbox_out.at[b])
    pltpu.sync_copy(maskoutbuf, masks_out.at[b])


_sc_topk = functools.partial(
    pl.kernel,
    out_type=(
        jax.ShapeDtypeStruct((B, NPAD), jnp.float32),
        jax.ShapeDtypeStruct((B, NPAD), jnp.int32),
        jax.ShapeDtypeStruct((B, 1200), jnp.float32),
        jax.ShapeDtypeStruct((B, NPAD), jnp.int32),
    ),
    mesh=plsc.VectorSubcoreMesh(core_axis_name="c", subcore_axis_name="s"),
    compiler_params=pltpu.CompilerParams(needs_layout_passes=False),
    scratch_types=[
        pltpu.VMEM((CHUNK,), jnp.float32),
        pltpu.VMEM((2049 * 16,), jnp.int32),
        pltpu.VMEM((320,), jnp.int32),
        pltpu.VMEM((320,), jnp.int32),
        pltpu.VMEM((CAP,), jnp.int32),
        pltpu.VMEM((CAP,), jnp.int32),
        pltpu.VMEM((NPAD,), jnp.int32),
        pltpu.VMEM((NPAD,), jnp.int32),
        pltpu.VMEM((NPAD,), jnp.float32),
        pltpu.VMEM((NPAD,), jnp.int32),
        pltpu.VMEM((NPAD,), jnp.int32),
        pltpu.VMEM((1200,), jnp.float32),
        pltpu.VMEM((4000,), jnp.float32),
        pltpu.VMEM((1024,), jnp.int32),
    ],
)(_sc_body)


@jax.jit
def kernel(logits, bbox, mask):
    bsz, q, c = logits.shape
    lg = logits.reshape(bsz * NCHUNK, CHUNK)
    bb = bbox.reshape(bsz, q * 4)
    mk = jnp.pad(mask.astype(jnp.int32), ((0, 0), (0, 1024 - q)))
    probs, caps, bbf, mko = _sc_topk(lg, bb, mk)
    return (
        probs[:, :NSEL],
        caps[:, :NSEL],
        bbf.reshape(bsz, NSEL, 4),
        mko[:, :NSEL].astype(bool),
    )


# speedup vs baseline: 1.0686x; 1.0686x over previous
import functools

import jax
import jax.numpy as jnp
from jax import lax
from jax.experimental import pallas as pl
from jax.experimental.pallas import tpu as pltpu
from jax.experimental.pallas import tpu_sc as plsc

NSEL = 300
NPAD = 304
B = 32
N = 256000
CHUNK = 32000
NCHUNK = N // CHUNK
VEC_PER_CHUNK = CHUNK // 16
H0 = 2048
H1 = 2048
H2 = 1024
INT_MIN = -2147483648
CAP = 4096


def _iota16():
    return lax.broadcasted_iota(jnp.int32, (16,), 0)


def _splat_last(v):
    return lax.gather(
        v,
        jnp.full((16, 1), 15, jnp.int32),
        lax.GatherDimensionNumbers(
            offset_dims=(), collapsed_slice_dims=(0,), start_index_map=(0,)),
        (1,),
        mode=lax.GatherScatterMode.PROMISE_IN_BOUNDS,
    )


def _to_key(x_f32):
    s = plsc.bitcast(x_f32, jnp.int32)
    m = lax.shift_right_arithmetic(s, 31)
    return s ^ lax.shift_right_logical(m, 1)


def _from_key(k_i32):
    s = k_i32 ^ lax.shift_right_logical(lax.shift_right_arithmetic(k_i32, 31), 1)
    return plsc.bitcast(s, jnp.float32)


def _zero_hist(hist, nbins):
    def body(r, _):
        hist[pl.ds(r * 16, 16)] = jnp.zeros((16,), jnp.int32)
        return 0

    lax.fori_loop(0, nbins, body, 0)


def _digit_select(hist, nbins, needed):
    def body(t, carry):
        suffix, b, nab = carry
        d = (nbins - 1) - t
        row = hist[pl.ds(d * 16, 16)]
        new_suffix = suffix + jnp.sum(row)
        hit = (suffix < needed) & (new_suffix >= needed)
        b = jnp.where(hit, d, b)
        nab = jnp.where(hit, suffix, nab)
        return (new_suffix, b, nab)

    _, b, nab = lax.fori_loop(0, nbins, body, (jnp.int32(0), jnp.int32(0), jnp.int32(0)))
    return b, nab


def _sc_body(lg_hbm, bbox_hbm, mask_hbm, probs_out, caps_out, bbox_out, masks_out,
             buf, buf2, hist, selk, seli, candk, candi, sortk, sorti,
             probbuf, capbuf, maskoutbuf, bboxoutbuf, bboxbuf, maskbuf,
             sem0, sem1):
    nc = 2
    b = lax.axis_index("s") * nc + lax.axis_index("c")
    lane = _iota16()
    ones = jnp.ones((16,), jnp.int32)

    pltpu.sync_copy(bbox_hbm.at[b], bboxbuf)
    pltpu.sync_copy(mask_hbm.at[b], maskbuf)

    _zero_hist(hist, H0)
    bufs = (buf, buf2)
    sems = (sem0, sem1)
    cp = [None, None]
    cp[0] = pltpu.async_copy(lg_hbm.at[b * NCHUNK], bufs[0], sems[0])
    for c in range(NCHUNK):
        if c + 1 < NCHUNK:
            cp[(c + 1) % 2] = pltpu.async_copy(
                lg_hbm.at[b * NCHUNK + c + 1], bufs[(c + 1) % 2], sems[(c + 1) % 2])
        cp[c % 2].wait()

        def p1(i, _, cbuf=bufs[c % 2]):
            base = i * 128
            for u in range(8):
                key = _to_key(cbuf[pl.ds(base + u * 16, 16)])
                d0 = lax.shift_right_logical(key ^ INT_MIN, 21)
                plsc.addupdate_scatter(hist, [lax.shift_left(d0, 4) | lane], ones)
            return 0

        lax.fori_loop(0, VEC_PER_CHUNK // 8, p1, 0)
    b0, n_above0 = _digit_select(hist, H0, jnp.int32(NSEL))

    g_s = jnp.zeros((16,), jnp.int32)
    nb_s = jnp.zeros((16,), jnp.int32)
    cp[0] = pltpu.async_copy(lg_hbm.at[b * NCHUNK], bufs[0], sems[0])
    for c in range(NCHUNK):
        if c + 1 < NCHUNK:
            cp[(c + 1) % 2] = pltpu.async_copy(
                lg_hbm.at[b * NCHUNK + c + 1], bufs[(c + 1) % 2], sems[(c + 1) % 2])
        cp[c % 2].wait()

        def p2(i, carry, cbuf=bufs[c % 2]):
            g_s, nb_s = carry
            base = i * 64
            for u in range(4):
                key = _to_key(cbuf[pl.ds(base + u * 16, 16)])
                kb = key ^ INT_MIN
                idx = c * CHUNK + base + u * 16 + lane
                d0 = lax.shift_right_logical(kb, 21)
                m_gt = d0 > b0
                run = g_s + plsc.cumsum(m_gt.astype(jnp.int32))
                pos = jnp.clip(run - 1, 0, 319)
                plsc.store_scatter(selk, [pos], key, mask=m_gt)
                plsc.store_scatter(seli, [pos], idx, mask=m_gt)
                g_s = _splat_last(run)
                m_b = d0 == b0
                run_b = nb_s + plsc.cumsum(m_b.astype(jnp.int32))
                pos_b = jnp.clip(run_b - 1, 0, CAP - 1)
                plsc.store_scatter(candk, [pos_b], key, mask=m_b)
                plsc.store_scatter(candi, [pos_b], idx, mask=m_b)
                nb_s = _splat_last(run_b)
            return (g_s, nb_s)

        g_s, nb_s = lax.fori_loop(0, VEC_PER_CHUNK // 4, p2, (g_s, nb_s))

    g = jnp.max(g_s)
    nb = jnp.max(nb_s)

    needed1 = jnp.int32(NSEL) - n_above0

    @pl.when(nb <= CAP)
    def _refine_local():
        _zero_hist(hist, H1)

        def r1(i, _):
            valid = (i * 16 + lane) < nb
            kb = candk[pl.ds(i * 16, 16)] ^ INT_MIN
            d1 = lax.shift_right_logical(kb, 10) & 0x7FF
            plsc.addupdate_scatter(hist, [lax.shift_left(d1, 4) | lane], ones, mask=valid)
            return 0

        lax.fori_loop(0, CAP // 16, r1, 0)
        b1, gt1 = _digit_select(hist, H1, needed1)
        n_above1 = n_above0 + gt1

        _zero_hist(hist, H2)

        def r2(i, gg):
            valid = (i * 16 + lane) < nb
            key = candk[pl.ds(i * 16, 16)]
            idx = candi[pl.ds(i * 16, 16)]
            kb = key ^ INT_MIN
            d1 = lax.shift_right_logical(kb, 10) & 0x7FF
            m_gt = valid & (d1 > b1)
            csum = plsc.cumsum(m_gt.astype(jnp.int32))
            pos = jnp.clip(gg + csum - 1, 0, 319)
            plsc.store_scatter(selk, [pos], key, mask=m_gt)
            plsc.store_scatter(seli, [pos], idx, mask=m_gt)
            m_eq = valid & (d1 == b1)
            d2 = kb & 0x3FF
            plsc.addupdate_scatter(hist, [lax.shift_left(d2, 4) | lane], ones, mask=m_eq)
            return gg + jnp.sum(m_gt.astype(jnp.int32))

        gg = lax.fori_loop(0, CAP // 16, r2, g)
        b2, gt2 = _digit_select(hist, H2, jnp.int32(NSEL) - n_above1)
        n_gt = n_above1 + gt2
        thr = (lax.shift_left(b0, 21) | lax.shift_left(b1, 10) | b2) ^ INT_MIN

        def r3(i, carry):
            gg, e = carry
            valid = (i * 16 + lane) < nb
            key = candk[pl.ds(i * 16, 16)]
            idx = candi[pl.ds(i * 16, 16)]
            kb = key ^ INT_MIN
            d1 = lax.shift_right_logical(kb, 10) & 0x7FF
            m_gt = valid & (d1 == b1) & (key > thr)
            csum = plsc.cumsum(m_gt.astype(jnp.int32))
            pos = jnp.clip(gg + csum - 1, 0, 319)
            plsc.store_scatter(selk, [pos], key, mask=m_gt)
            plsc.store_scatter(seli, [pos], idx, mask=m_gt)
            gg = gg + jnp.sum(m_gt.astype(jnp.int32))
            m_eq = valid & (key == thr)
            csum_e = plsc.cumsum(m_eq.astype(jnp.int32))
            take = m_eq & ((e + csum_e) <= NSEL)
            pos_e = jnp.clip(e + csum_e - 1, 0, 319)
            plsc.store_scatter(selk, [pos_e], key, mask=take)
            plsc.store_scatter(seli, [pos_e], idx, mask=take)
            return (gg, e + jnp.sum(take.astype(jnp.int32)))

        lax.fori_loop(0, CAP // 16, r3, (n_above1, n_gt))

    @pl.when(nb > CAP)
    def _refine_streamed():
        _zero_hist(hist, H1)
        for c in range(NCHUNK):
            pltpu.sync_copy(lg_hbm.at[b * NCHUNK + c], buf)

            def f1(i, _):
                kb = _to_key(buf[pl.ds(i * 16, 16)]) ^ INT_MIN
                d0 = lax.shift_right_logical(kb, 21)
                d1 = lax.shift_right_logical(kb, 10) & 0x7FF
                plsc.addupdate_scatter(hist, [lax.shift_left(d1, 4) | lane], ones,
                                       mask=d0 == b0)
                return 0

            lax.fori_loop(0, VEC_PER_CHUNK, f1, 0)
        b1, gt1 = _digit_select(hist, H1, needed1)
        n_above1 = n_above0 + gt1
        p21 = lax.shift_left(b0, 11) | b1

        _zero_hist(hist, H2)
        gg = g
        for c in range(NCHUNK):
            pltpu.sync_copy(lg_hbm.at[b * NCHUNK + c], buf)

            def f2(i, gg):
                key = _to_key(buf[pl.ds(i * 16, 16)])
                kb = key ^ INT_MIN
                idx = c * CHUNK + i * 16 + lane
                pfx = lax.shift_right_logical(kb, 10)
                m_gt = (pfx > p21) & (lax.shift_right_logical(kb, 21) == b0)
                csum = plsc.cumsum(m_gt.astype(jnp.int32))
                pos = jnp.clip(gg + csum - 1, 0, 319)
                plsc.store_scatter(selk, [pos], key, mask=m_gt)
                plsc.store_scatter(seli, [pos], idx, mask=m_gt)
                m_eq = pfx == p21
                d2 = kb & 0x3FF
                plsc.addupdate_scatter(hist, [lax.shift_left(d2, 4) | lane], ones, mask=m_eq)
                return gg + jnp.sum(m_gt.astype(jnp.int32))

            gg = lax.fori_loop(0, VEC_PER_CHUNK, f2, gg)
        b2, gt2 = _digit_select(hist, H2, jnp.int32(NSEL) - n_above1)
        n_gt = n_above1 + gt2
        thr = (lax.shift_left(p21, 10) | b2) ^ INT_MIN

        e = n_gt
        for c in range(NCHUNK):
            pltpu.sync_copy(lg_hbm.at[b * NCHUNK + c], buf)

            def f3(i, carry):
                gg, e = carry
                key = _to_key(buf[pl.ds(i * 16, 16)])
                kb = key ^ INT_MIN
                idx = c * CHUNK + i * 16 + lane
                pfx = lax.shift_right_logical(kb, 10)
                m_gt = (pfx == p21) & (key > thr)
                csum = plsc.cumsum(m_gt.astype(jnp.int32))
                pos = jnp.clip(gg + csum - 1, 0, 319)
                plsc.store_scatter(selk, [pos], key, mask=m_gt)
                plsc.store_scatter(seli, [pos], idx, mask=m_gt)
                gg = gg + jnp.sum(m_gt.astype(jnp.int32))
                m_eq = key == thr
                csum_e = plsc.cumsum(m_eq.astype(jnp.int32))
                take = m_eq & ((e + csum_e) <= NSEL)
                pos_e = jnp.clip(e + csum_e - 1, 0, 319)
                plsc.store_scatter(selk, [pos_e], key, mask=take)
                plsc.store_scatter(seli, [pos_e], idx, mask=take)
                return (gg, e + jnp.sum(take.astype(jnp.int32)))

            gg, e = lax.fori_loop(0, VEC_PER_CHUNK, f3, (gg, e))

    cur_k = selk[pl.ds(288, 16)]
    cur_i = seli[pl.ds(288, 16)]
    m_pad = lane >= 12
    selk[pl.ds(288, 16)] = jnp.where(m_pad, jnp.full((16,), INT_MIN, jnp.int32), cur_k)
    seli[pl.ds(288, 16)] = jnp.where(m_pad, 0x4000000 + lane, cur_i)

    sortk[pl.ds(288, 16)] = jnp.zeros((16,), jnp.int32)
    sorti[pl.ds(288, 16)] = jnp.zeros((16,), jnp.int32)

    for iv in range(NPAD // 16):
        ki = selk[pl.ds(iv * 16, 16)]
        ii = seli[pl.ds(iv * 16, 16)]

        def rank_body(j, rank):
            jb = jnp.full((16,), j, jnp.int32)
            kj = plsc.load_gather(selk, [jb])
            ij = plsc.load_gather(seli, [jb])
            beats = (kj > ki) | ((kj == ki) & (ij < ii))
            return rank + beats.astype(jnp.int32)

        rank = lax.fori_loop(0, NPAD, rank_body, jnp.zeros((16,), jnp.int32))
        m_real = rank < NSEL
        plsc.store_scatter(sortk, [rank], ki, mask=m_real)
        plsc.store_scatter(sorti, [rank], ii, mask=m_real)

    for v in range(NPAD // 16):
        sl = pl.ds(v * 16, 16)
        k = sortk[sl]
        idx = sorti[sl]
        logit = _from_key(k)
        probbuf[sl] = 1.0 / (1.0 + jnp.exp(-logit))
        capbuf[sl] = idx & 255
        box = jnp.minimum(lax.shift_right_logical(idx, 8), 999)
        maskoutbuf[sl] = plsc.load_gather(maskbuf, [box])

    for v in range(1200 // 16):
        bi = v * 4 + lax.shift_right_logical(lane, 2)
        si = plsc.load_gather(sorti, [bi])
        box = jnp.minimum(lax.shift_right_logical(si, 8), 999)
        addr = lax.shift_left(box, 2) | (lane & 3)
        bboxoutbuf[pl.ds(v * 16, 16)] = plsc.load_gather(bboxbuf, [addr])

    pltpu.sync_copy(probbuf, probs_out.at[b])
    pltpu.sync_copy(capbuf, caps_out.at[b])
    pltpu.sync_copy(bboxoutbuf, bbox_out.at[b])
    pltpu.sync_copy(maskoutbuf, masks_out.at[b])


_sc_topk = functools.partial(
    pl.kernel,
    out_type=(
        jax.ShapeDtypeStruct((B, NPAD), jnp.float32),
        jax.ShapeDtypeStruct((B, NPAD), jnp.int32),
        jax.ShapeDtypeStruct((B, 1200), jnp.float32),
        jax.ShapeDtypeStruct((B, NPAD), jnp.int32),
    ),
    mesh=plsc.VectorSubcoreMesh(core_axis_name="c", subcore_axis_name="s"),
    compiler_params=pltpu.CompilerParams(needs_layout_passes=False),
    scratch_types=[
        pltpu.VMEM((CHUNK,), jnp.float32),
        pltpu.VMEM((CHUNK,), jnp.float32),
        pltpu.VMEM((H0 * 16,), jnp.int32),
        pltpu.VMEM((320,), jnp.int32),
        pltpu.VMEM((320,), jnp.int32),
        pltpu.VMEM((CAP,), jnp.int32),
        pltpu.VMEM((CAP,), jnp.int32),
        pltpu.VMEM((NPAD,), jnp.int32),
        pltpu.VMEM((NPAD,), jnp.int32),
        pltpu.VMEM((NPAD,), jnp.float32),
        pltpu.VMEM((NPAD,), jnp.int32),
        pltpu.VMEM((NPAD,), jnp.int32),
        pltpu.VMEM((1200,), jnp.float32),
        pltpu.VMEM((4000,), jnp.float32),
        pltpu.VMEM((1024,), jnp.int32),
        pltpu.SemaphoreType.DMA,
        pltpu.SemaphoreType.DMA,
    ],
)(_sc_body)


@jax.jit
def kernel(logits, bbox, mask):
    bsz, q, c = logits.shape
    lg = logits.reshape(bsz * NCHUNK, CHUNK)
    bb = bbox.reshape(bsz, q * 4)
    mk = jnp.pad(mask.astype(jnp.int32), ((0, 0), (0, 1024 - q)))
    probs, caps, bbf, mko = _sc_topk(lg, bb, mk)
    return (
        probs[:, :NSEL],
        caps[:, :NSEL],
        bbf.reshape(bsz, NSEL, 4),
        mko[:, :NSEL].astype(bool),
    )
